# baseline (speedup 1.0000x reference)
import functools

import jax
import jax.numpy as jnp
from jax import lax
from jax.experimental import pallas as pl
from jax.experimental.pallas import tpu as pltpu
from jax.experimental.pallas import tpu_sc as plsc

B, D_IN, T = 4, 1024, 4096
CB_SIZE, CB_DIM = 8192, 8
CB_PAD = 128

TN = 512
TN2 = 512


def _encode_body(z_ref, w_in_ref, in_b_ref, cb_n_ref, cb2_ref, ze_ref, idx_ref):
    z_blk = z_ref[0]
    w_in = w_in_ref[...]
    z_e = jnp.dot(w_in, z_blk, preferred_element_type=jnp.float32)
    z_e = z_e + in_b_ref[...]
    ze_ref[0] = z_e

    nrm = jnp.sqrt(jnp.sum(z_e * z_e, axis=0, keepdims=True))
    en = z_e / jnp.maximum(nrm, 1e-12)
    en2 = jnp.sum(en * en, axis=0, keepdims=True)

    m = jnp.dot(cb_n_ref[...], en, preferred_element_type=jnp.float32)
    dist = (en2 - 2.0 * m) + cb2_ref[...]

    half = CB_SIZE // 2
    iota = lax.broadcasted_iota(jnp.int32, (half, dist.shape[1]), 0)

    def half_argmin(vh, base):
        mn = jnp.min(vh, axis=0, keepdims=True)
        cand = jnp.where(vh == mn, iota, CB_SIZE)
        return mn, jnp.min(cand, axis=0, keepdims=True) + base

    v0, i0 = half_argmin(dist[:half], 0)
    v1, i1 = half_argmin(dist[half:], half)
    bv0 = v0.astype(jnp.bfloat16).astype(jnp.float32)
    idx = jnp.where(v1 < bv0, i1, i0)
    idx_ref[0] = jnp.broadcast_to(idx, (CB_DIM, idx.shape[1]))


def _encode(z, w_in, in_b, cb_n, cb2, t_blk_off=0, t_len=T):
    grid = (B, t_len // TN)
    return pl.pallas_call(
        _encode_body,
        grid=grid,
        in_specs=[
            pl.BlockSpec((1, D_IN, TN), lambda b, t: (b, 0, t + t_blk_off)),
            pl.BlockSpec((CB_DIM, D_IN), lambda b, t: (0, 0)),
            pl.BlockSpec((CB_DIM, 1), lambda b, t: (0, 0)),
            pl.BlockSpec((CB_SIZE, CB_DIM), lambda b, t: (0, 0)),
            pl.BlockSpec((CB_SIZE, 1), lambda b, t: (0, 0)),
        ],
        out_specs=[
            pl.BlockSpec((1, CB_DIM, TN), lambda b, t: (b, 0, t)),
            pl.BlockSpec((1, CB_DIM, TN), lambda b, t: (b, 0, t)),
        ],
        out_shape=[
            jax.ShapeDtypeStruct((B, CB_DIM, t_len), jnp.float32),
            jax.ShapeDtypeStruct((B, CB_DIM, t_len), jnp.int32),
        ],
    )(z, w_in, in_b, cb_n, cb2)


_IDX_CHUNK = 128


def _make_sc_gather(n_rows):
    info = plsc.get_sparse_core_info()
    nw = info.num_cores * info.num_subcores
    b_per_w = n_rows // nw
    n_chunks = b_per_w // _IDX_CHUNK

    mesh = plsc.VectorSubcoreMesh(core_axis_name="c", subcore_axis_name="s")

    @functools.partial(
        pl.kernel,
        mesh=mesh,
        out_type=jax.ShapeDtypeStruct((n_rows, CB_PAD), jnp.float32),
        scratch_types=[
            pltpu.VMEM((n_chunks, _IDX_CHUNK), jnp.int32),
            pltpu.VMEM((b_per_w, CB_PAD), jnp.float32),
            pltpu.SemaphoreType.DMA,
        ],
    )
    def gather(table_hbm, idx_hbm, out_hbm, idx_v, rows_v, sem):
        wid = lax.axis_index("s") * info.num_cores + lax.axis_index("c")
        pltpu.sync_copy(idx_hbm.at[wid], idx_v)
        copies = []
        for j in range(n_chunks):
            copies.append(
                pltpu.async_copy(
                    table_hbm.at[idx_v.at[j]],
                    rows_v.at[pl.ds(j * _IDX_CHUNK, _IDX_CHUNK)],
                    sem,
                )
            )
        for c in copies:
            c.wait()
        pltpu.sync_copy(rows_v, out_hbm.at[pl.ds(wid * b_per_w, b_per_w)])

    return gather


def _decode_body(zq_ref, w_out_ref, out_b_ref, out_ref):
    zq = zq_ref[0]
    w_out = w_out_ref[...]
    out = lax.dot_general(w_out, zq, (((1,), (1,)), ((), ())),
                          preferred_element_type=jnp.float32)
    out_ref[0] = out + out_b_ref[...]


def _decode(zq3, w_out_p, out_b):
    grid = (B, T // TN2)
    return pl.pallas_call(
        _decode_body,
        grid=grid,
        in_specs=[
            pl.BlockSpec((1, TN2, CB_PAD), lambda b, t: (b, t, 0)),
            pl.BlockSpec((D_IN, CB_PAD), lambda b, t: (0, 0)),
            pl.BlockSpec((D_IN, 1), lambda b, t: (0, 0)),
        ],
        out_specs=pl.BlockSpec((1, D_IN, TN2), lambda b, t: (b, 0, t)),
        out_shape=jax.ShapeDtypeStruct((B, D_IN, T), jnp.float32),
    )(zq3, w_out_p, out_b)


def _weight_norm(v, g):
    norm = jnp.sqrt(jnp.sum(v * v, axis=1, keepdims=True))
    return g[:, None] * v / norm


def kernel(z, in_v, in_g, in_b, out_v, out_g, out_b, codebook):
    w_in = _weight_norm(in_v, in_g)
    w_out = _weight_norm(out_v, out_g)
    cb_nrm = jnp.sqrt(jnp.sum(codebook * codebook, axis=1, keepdims=True))
    cb_n = codebook / jnp.maximum(cb_nrm, 1e-12)
    cb2 = jnp.sum(cb_n * cb_n, axis=1, keepdims=True)
    cb_pad = jnp.pad(codebook, ((0, 0), (0, CB_PAD - CB_DIM)))
    w_out_p = jnp.pad(w_out, ((0, 0), (0, CB_PAD - CB_DIM)))

    TH = T // 2
    info = plsc.get_sparse_core_info()
    nw = info.num_cores * info.num_subcores
    n_rows = B * TH

    def gather_half(idx_half):
        grouped = idx_half.reshape(nw, (n_rows // nw) // _IDX_CHUNK, _IDX_CHUNK)
        return _make_sc_gather(n_rows)(cb_pad, grouped)

    ze0, idxb0 = _encode(z, w_in, in_b[:, None], cb_n, cb2, 0, TH)
    zq0 = gather_half(idxb0[:, 0, :].reshape(-1))
    ze1, idxb1 = _encode(z, w_in, in_b[:, None], cb_n, cb2, TH // TN, TH)
    zq1 = gather_half(idxb1[:, 0, :].reshape(-1))

    z_e = jnp.concatenate([ze0, ze1], axis=2)
    indices = jnp.concatenate([idxb0[:, 0, :], idxb1[:, 0, :]], axis=1)
    zq3 = jnp.concatenate(
        [zq0.reshape(B, TH, CB_PAD), zq1.reshape(B, TH, CB_PAD)], axis=1)
    z_q_out = _decode(zq3, w_out_p, out_b[:, None])

    zeros = jnp.zeros((B,), dtype=jnp.float32)
    return (z_q_out, zeros, zeros, indices, z_e)


# speedup vs baseline: 1.0678x; 1.0678x over previous
import functools

import jax
import jax.numpy as jnp
from jax import lax
from jax.experimental import pallas as pl
from jax.experimental.pallas import tpu as pltpu
from jax.experimental.pallas import tpu_sc as plsc

B, D_IN, T = 4, 1024, 4096
CB_SIZE, CB_DIM = 8192, 8
CB_PAD = 128

TN = 1024
TN2 = 512


def _encode_body(z_ref, w_in_ref, in_b_ref, cb_n_ref, cb2_ref, ze_ref, idx_ref):
    z_blk = z_ref[0]
    w_in = w_in_ref[...]
    z_e = jnp.dot(w_in, z_blk, preferred_element_type=jnp.float32)
    z_e = z_e + in_b_ref[...]
    ze_ref[0] = z_e

    nrm = jnp.sqrt(jnp.sum(z_e * z_e, axis=0, keepdims=True))
    en = z_e / jnp.maximum(nrm, 1e-12)
    en2 = jnp.sum(en * en, axis=0, keepdims=True)

    m = jnp.dot(cb_n_ref[...], en, preferred_element_type=jnp.float32)
    dist = (en2 - 2.0 * m) + cb2_ref[...]

    half = CB_SIZE // 2
    iota = lax.broadcasted_iota(jnp.int32, (half, dist.shape[1]), 0)

    def half_argmin(vh, base):
        mn = jnp.min(vh, axis=0, keepdims=True)
        cand = jnp.where(vh == mn, iota, CB_SIZE)
        return mn, jnp.min(cand, axis=0, keepdims=True) + base

    v0, i0 = half_argmin(dist[:half], 0)
    v1, i1 = half_argmin(dist[half:], half)
    bv0 = v0.astype(jnp.bfloat16).astype(jnp.float32)
    idx = jnp.where(v1 < bv0, i1, i0)
    idx_ref[0] = jnp.broadcast_to(idx, (CB_DIM, idx.shape[1]))


def _encode(z, w_in, in_b, cb_n, cb2):
    grid = (B, T // TN)
    return pl.pallas_call(
        _encode_body,
        grid=grid,
        in_specs=[
            pl.BlockSpec((1, D_IN, TN), lambda b, t: (b, 0, t)),
            pl.BlockSpec((CB_DIM, D_IN), lambda b, t: (0, 0)),
            pl.BlockSpec((CB_DIM, 1), lambda b, t: (0, 0)),
            pl.BlockSpec((CB_SIZE, CB_DIM), lambda b, t: (0, 0)),
            pl.BlockSpec((CB_SIZE, 1), lambda b, t: (0, 0)),
        ],
        out_specs=[
            pl.BlockSpec((1, CB_DIM, TN), lambda b, t: (b, 0, t)),
            pl.BlockSpec((1, CB_DIM, TN), lambda b, t: (b, 0, t)),
        ],
        out_shape=[
            jax.ShapeDtypeStruct((B, CB_DIM, T), jnp.float32),
            jax.ShapeDtypeStruct((B, CB_DIM, T), jnp.int32),
        ],
    )(z, w_in, in_b, cb_n, cb2)


_IDX_CHUNK = 128


def _make_sc_gather(n_rows):
    info = plsc.get_sparse_core_info()
    nw = info.num_cores * info.num_subcores
    b_per_w = n_rows // nw
    n_chunks = b_per_w // _IDX_CHUNK

    mesh = plsc.VectorSubcoreMesh(core_axis_name="c", subcore_axis_name="s")

    @functools.partial(
        pl.kernel,
        mesh=mesh,
        out_type=jax.ShapeDtypeStruct((n_rows, CB_PAD), jnp.float32),
        scratch_types=[
            pltpu.VMEM((n_chunks, _IDX_CHUNK), jnp.int32),
            pltpu.VMEM((b_per_w, CB_PAD), jnp.float32),
            pltpu.SemaphoreType.DMA,
        ],
    )
    def gather(table_hbm, idx_hbm, out_hbm, idx_v, rows_v, sem):
        wid = lax.axis_index("s") * info.num_cores + lax.axis_index("c")
        pltpu.sync_copy(idx_hbm.at[wid], idx_v)
        copies = []
        for j in range(n_chunks):
            copies.append(
                pltpu.async_copy(
                    table_hbm.at[idx_v.at[j]],
                    rows_v.at[pl.ds(j * _IDX_CHUNK, _IDX_CHUNK)],
                    sem,
                )
            )
        for c in copies:
            c.wait()
        pltpu.sync_copy(rows_v, out_hbm.at[pl.ds(wid * b_per_w, b_per_w)])

    return gather


def _decode_body(zq_ref, w_out_ref, out_b_ref, out_ref):
    zq = zq_ref[0]
    w_out = w_out_ref[...]
    out = lax.dot_general(w_out, zq, (((1,), (1,)), ((), ())),
                          preferred_element_type=jnp.float32)
    out_ref[0] = out + out_b_ref[...]


def _decode(zq3, w_out_p, out_b):
    grid = (B, T // TN2)
    return pl.pallas_call(
        _decode_body,
        grid=grid,
        in_specs=[
            pl.BlockSpec((1, TN2, CB_PAD), lambda b, t: (b, t, 0)),
            pl.BlockSpec((D_IN, CB_PAD), lambda b, t: (0, 0)),
            pl.BlockSpec((D_IN, 1), lambda b, t: (0, 0)),
        ],
        out_specs=pl.BlockSpec((1, D_IN, TN2), lambda b, t: (b, 0, t)),
        out_shape=jax.ShapeDtypeStruct((B, D_IN, T), jnp.float32),
    )(zq3, w_out_p, out_b)


def _weight_norm(v, g):
    norm = jnp.sqrt(jnp.sum(v * v, axis=1, keepdims=True))
    return g[:, None] * v / norm


def kernel(z, in_v, in_g, in_b, out_v, out_g, out_b, codebook):
    w_in = _weight_norm(in_v, in_g)
    w_out = _weight_norm(out_v, out_g)
    cb_nrm = jnp.sqrt(jnp.sum(codebook * codebook, axis=1, keepdims=True))
    cb_n = codebook / jnp.maximum(cb_nrm, 1e-12)
    cb2 = jnp.sum(cb_n * cb_n, axis=1, keepdims=True)
    cb_pad = jnp.pad(codebook, ((0, 0), (0, CB_PAD - CB_DIM)))
    w_out_p = jnp.pad(w_out, ((0, 0), (0, CB_PAD - CB_DIM)))

    z_e, idx_bcast = _encode(z, w_in, in_b[:, None], cb_n, cb2)
    indices = idx_bcast[:, 0, :]

    idx_flat = indices.reshape(-1)
    n_rows = B * T
    info = plsc.get_sparse_core_info()
    nw = info.num_cores * info.num_subcores
    idx_grouped = idx_flat.reshape(nw, (n_rows // nw) // _IDX_CHUNK, _IDX_CHUNK)
    zq_rows = _make_sc_gather(n_rows)(cb_pad, idx_grouped)

    z_q_out = _decode(zq_rows.reshape(B, T, CB_PAD), w_out_p, out_b[:, None])

    zeros = jnp.zeros((B,), dtype=jnp.float32)
    return (z_q_out, zeros, zeros, indices, z_e)
